# baseline (speedup 1.0000x reference)
import functools

import jax
import jax.numpy as jnp
from jax import lax
from jax.experimental import pallas as pl
from jax.experimental.pallas import tpu as pltpu
from jax.experimental.pallas import tpu_sc as plsc

_D = 32
_B = 16384
_F = 26
_NW = 32
_C0 = 15625
_C1 = 65536
_CC = 81280
_VB = _CC // 128
_BT = _B // 128
_UNITS = _F * _BT
_UPW = _UNITS // _NW
_BPW = 4

_mesh = plsc.VectorSubcoreMesh(core_axis_name="c", subcore_axis_name="s")


@functools.partial(
    pl.kernel,
    out_type=jax.ShapeDtypeStruct((_CC * _D // 128, 128), jnp.float32),
    mesh=_mesh,
    compiler_params=pltpu.CompilerParams(use_tc_tiling_on_sc=True, needs_layout_passes=False, disable_bounds_checks=True),
    scratch_types=[
        pltpu.VMEM((2, 4, 8, 128), jnp.float32),
        pltpu.VMEM((2, 32, 128), jnp.float32),
        pltpu.SemaphoreType.DMA,
        pltpu.SemaphoreType.DMA,
        pltpu.SemaphoreType.DMA,
        pltpu.SemaphoreType.DMA,
    ],
)
def _relayout(src_hbm, lin_hbm, vin, vout, gsem0, gsem1, osem0, osem1):
    cid = lax.axis_index("c")
    sid = lax.axis_index("s")
    wid = sid * 2 + cid
    gsems = (gsem0, gsem1)
    osems = (osem0, osem1)
    nfull = 19

    def fire(n):
        p = n % 2
        vb = wid + 32 * n
        for dt in range(4):
            pltpu.async_copy(
                src_hbm.at[pl.ds(dt * 8, 8), pl.ds(vb * 128, 128)],
                vin.at[p, dt], gsems[p])

    def retire(n):
        p = n % 2
        vb = wid + 32 * n
        for dt in range(4):
            pltpu.make_async_copy(lin_hbm.at[pl.ds(0, 8)], vin.at[p, dt],
                                  gsems[p]).wait()
        if n >= 2:
            pltpu.make_async_copy(lin_hbm.at[pl.ds(0, 32)], vout.at[p],
                                  osems[p]).wait()

        @plsc.parallel_loop(0, 256, unroll=4)
        def _asm(j):
            mr = lax.shift_right_logical(j, 3)
            jj = lax.bitwise_and(j, 7)
            lane0 = jj * 16
            d = lax.bitwise_and(lane0, 31) + lax.iota(jnp.int32, 16)
            dt_v = lax.shift_right_logical(d, 3)
            di_v = lax.bitwise_and(d, 7)
            vi = jnp.full((16,), 0, jnp.int32) + (mr * 4 + lax.shift_right_logical(jj, 1))
            vout[p, mr, pl.ds(lane0, 16)] = plsc.load_gather(
                vin.at[p], [dt_v, di_v, vi])

        pltpu.async_copy(vout.at[p], lin_hbm.at[pl.ds(vb * 32, 32)], osems[p])

    fire(0)
    for n in range(1, nfull):
        fire(n)
        retire(n - 1)

    @pl.when(wid < 27)
    def _():
        fire(19)
    retire(nfull - 1)

    @pl.when(wid < 27)
    def _():
        retire(19)

    pltpu.make_async_copy(lin_hbm.at[pl.ds(0, 32)], vout.at[0],
                          osems[0]).wait()
    pltpu.make_async_copy(lin_hbm.at[pl.ds(0, 32)], vout.at[1],
                          osems[1]).wait()


@functools.partial(
    pl.kernel,
    out_type=jax.ShapeDtypeStruct((_F, 4, 128, 1024), jnp.float32),
    mesh=_mesh,
    compiler_params=pltpu.CompilerParams(use_tc_tiling_on_sc=False, needs_layout_passes=False, disable_bounds_checks=True),
    scratch_types=[
        pltpu.VMEM((_F, 512), jnp.int32),
        pltpu.VMEM((2, 512), jnp.int32),
        pltpu.VMEM((2, 512), jnp.int32),
        pltpu.VMEM((2, 512, _D), jnp.float32),
        pltpu.VMEM((2, 512, _D), jnp.float32),
        pltpu.VMEM((2, 4, 4, 1024), jnp.float32),
        pltpu.SemaphoreType.DMA,
        pltpu.SemaphoreType.DMA,
        pltpu.SemaphoreType.DMA,
        pltpu.SemaphoreType.DMA,
    ],
)
def _gather(ids_hbm, ctab_hbm, out_hbm, ids_v, idx0_v, idx1_v, buf0, buf1,
            vout, gsem0, gsem1, osem0, osem1):
    cid = lax.axis_index("c")
    sid = lax.axis_index("s")
    wid = sid * 2 + cid
    gsems = (gsem0, gsem1)
    osems = (osem0, osem1)

    pltpu.sync_copy(ids_hbm.at[:, pl.ds(wid * 512, 512)], ids_v)

    iota = lax.iota(jnp.int32, 16)
    cdt = tuple(lax.shift_right_logical(h * 16 + iota, 3) for h in range(2))
    clo = tuple(lax.bitwise_and(h * 16 + iota, 7) * 128 for h in range(2))

    def stage_a(f, p):
        @plsc.parallel_loop(0, 32, unroll=4)
        def _idx(i):
            sl = pl.ds(i * 16, 16)
            v = ids_v[f, sl]
            idx1_v[p, sl] = lax.bitwise_and(v, jnp.int32(65535)) + jnp.int32(_C0)
            h = lax.shift_right_logical(v, 16)
            x = h * jnp.int32(1024) + jnp.where(
                h >= jnp.int32(32768), jnp.int32(511), jnp.int32(0))
            q = (x.astype(jnp.float32) * jnp.float32(1.0 / 15625)).astype(jnp.int32)
            r = x - q * jnp.int32(15625)
            r = jnp.where(r < 0, r + jnp.int32(15625), r)
            r = jnp.where(r >= jnp.int32(15625), r - jnp.int32(15625), r)
            idx0_v[p, sl] = r

        for j in range(4):
            jsl = pl.ds(j * 128, 128)
            pltpu.async_copy(ctab_hbm.at[idx0_v.at[p, jsl]], buf0.at[p, jsl],
                             gsems[p])
            pltpu.async_copy(ctab_hbm.at[idx1_v.at[p, jsl]], buf1.at[p, jsl],
                             gsems[p])

    def stage_b(f, p, first):
        for j in range(4):
            jsl = pl.ds(j * 128, 128)
            pltpu.make_async_copy(ctab_hbm.at[pl.ds(0, 128)], buf0.at[p, jsl],
                                  gsems[p]).wait()
            pltpu.make_async_copy(ctab_hbm.at[pl.ds(0, 128)], buf1.at[p, jsl],
                                  gsems[p]).wait()

        def drain_out():
            for dt in range(4):
                pltpu.make_async_copy(out_hbm.at[0, 0, pl.ds(0, _BPW)],
                                      vout.at[p, dt], osems[p]).wait()

        if first is None:
            drain_out()
        else:
            pl.when(first)(drain_out)

        @plsc.parallel_loop(0, 512, unroll=4)
        def _asm(r_):
            btl = jnp.full((16,), 0, jnp.int32) + lax.shift_right_logical(r_, 7)
            bi = jnp.full((16,), 0, jnp.int32) + lax.bitwise_and(r_, 127)
            for half in range(2):
                sl = pl.ds(half * 16, 16)
                s = buf0[p, r_, sl] + buf1[p, r_, sl]
                plsc.store_scatter(vout.at[p], [cdt[half], btl, clo[half] + bi], s)

        for dt in range(4):
            pltpu.async_copy(vout.at[p, dt],
                             out_hbm.at[f, dt, pl.ds(wid * _BPW, _BPW)],
                             osems[p])

    stage_a(jnp.int32(0), 0)
    stage_a(jnp.int32(1), 1)

    def outer(go, carry):
        f0 = 2 * go
        for k in range(2):
            stage_b(f0 + k, k, go > 0)
            stage_a(f0 + k + 2, k)
        return carry

    lax.fori_loop(0, (_F - 2) // 2, outer, 0)

    stage_b(jnp.int32(24), 0, None)
    stage_b(jnp.int32(25), 1, None)
    for p in (0, 1):
        for dt in range(4):
            pltpu.make_async_copy(out_hbm.at[0, 0, pl.ds(0, _BPW)],
                                  vout.at[p, dt], osems[p]).wait()


def kernel(ids, table):
    c0 = table[::64]
    c1 = table[:_C1]
    pad = jnp.zeros((_CC - _C0 - _C1, _D), jnp.float32)
    cc = jnp.concatenate([c0, c1, pad], axis=0)
    lin = _relayout(cc.T)
    out4 = _gather(ids.T, lin.reshape(_CC, _D))
    out5 = out4.reshape(_F, 4, 128, 8, 128)
    return out5.transpose(2, 4, 0, 1, 3).reshape(_B, _F, _D)


# speedup vs baseline: 1.0021x; 1.0021x over previous
import functools

import jax
import jax.numpy as jnp
from jax import lax
from jax.experimental import pallas as pl
from jax.experimental.pallas import tpu as pltpu
from jax.experimental.pallas import tpu_sc as plsc

_D = 32
_B = 16384
_F = 26
_NW = 32
_C0 = 15625
_C1 = 65536
_CC = 81280
_VB = _CC // 128
_BT = _B // 128
_UNITS = _F * _BT
_UPW = _UNITS // _NW
_BPW = 4

_mesh = plsc.VectorSubcoreMesh(core_axis_name="c", subcore_axis_name="s")


@functools.partial(
    pl.kernel,
    out_type=jax.ShapeDtypeStruct((_CC * _D // 128, 128), jnp.float32),
    mesh=_mesh,
    compiler_params=pltpu.CompilerParams(use_tc_tiling_on_sc=True, needs_layout_passes=False, disable_bounds_checks=True),
    scratch_types=[
        pltpu.VMEM((2, 4, 8, 128), jnp.float32),
        pltpu.VMEM((2, 32, 128), jnp.float32),
        pltpu.SemaphoreType.DMA,
        pltpu.SemaphoreType.DMA,
        pltpu.SemaphoreType.DMA,
        pltpu.SemaphoreType.DMA,
    ],
)
def _relayout(src_hbm, lin_hbm, vin, vout, gsem0, gsem1, osem0, osem1):
    cid = lax.axis_index("c")
    sid = lax.axis_index("s")
    wid = sid * 2 + cid
    gsems = (gsem0, gsem1)
    osems = (osem0, osem1)
    nfull = 19

    def fire(n):
        p = n % 2
        vb = wid + 32 * n
        for dt in range(4):
            pltpu.async_copy(
                src_hbm.at[pl.ds(dt * 8, 8), pl.ds(vb * 128, 128)],
                vin.at[p, dt], gsems[p])

    def retire(n):
        p = n % 2
        vb = wid + 32 * n
        for dt in range(4):
            pltpu.make_async_copy(lin_hbm.at[pl.ds(0, 8)], vin.at[p, dt],
                                  gsems[p]).wait()
        if n >= 2:
            pltpu.make_async_copy(lin_hbm.at[pl.ds(0, 32)], vout.at[p],
                                  osems[p]).wait()

        @plsc.parallel_loop(0, 256, unroll=4)
        def _asm(j):
            mr = lax.shift_right_logical(j, 3)
            jj = lax.bitwise_and(j, 7)
            lane0 = jj * 16
            d = lax.bitwise_and(lane0, 31) + lax.iota(jnp.int32, 16)
            dt_v = lax.shift_right_logical(d, 3)
            di_v = lax.bitwise_and(d, 7)
            vi = jnp.full((16,), 0, jnp.int32) + (mr * 4 + lax.shift_right_logical(jj, 1))
            vout[p, mr, pl.ds(lane0, 16)] = plsc.load_gather(
                vin.at[p], [dt_v, di_v, vi])

        pltpu.async_copy(vout.at[p], lin_hbm.at[pl.ds(vb * 32, 32)], osems[p])

    fire(0)
    for n in range(1, nfull):
        fire(n)
        retire(n - 1)

    @pl.when(wid < 27)
    def _():
        fire(19)
    retire(nfull - 1)

    @pl.when(wid < 27)
    def _():
        retire(19)

    pltpu.make_async_copy(lin_hbm.at[pl.ds(0, 32)], vout.at[0],
                          osems[0]).wait()
    pltpu.make_async_copy(lin_hbm.at[pl.ds(0, 32)], vout.at[1],
                          osems[1]).wait()


@functools.partial(
    pl.kernel,
    out_type=jax.ShapeDtypeStruct((_F, 4, 128, 1024), jnp.float32),
    mesh=_mesh,
    compiler_params=pltpu.CompilerParams(use_tc_tiling_on_sc=False, needs_layout_passes=False, disable_bounds_checks=True),
    scratch_types=[
        pltpu.VMEM((_F, 512), jnp.int32),
        pltpu.VMEM((2, 512), jnp.int32),
        pltpu.VMEM((2, 512), jnp.int32),
        pltpu.VMEM((2, 512, _D), jnp.float32),
        pltpu.VMEM((2, 512, _D), jnp.float32),
        pltpu.VMEM((2, 4, 4, 1024), jnp.float32),
        pltpu.SemaphoreType.DMA,
        pltpu.SemaphoreType.DMA,
        pltpu.SemaphoreType.DMA,
        pltpu.SemaphoreType.DMA,
    ],
)
def _gather(ids_hbm, ctab_hbm, out_hbm, ids_v, idx0_v, idx1_v, buf0, buf1,
            vout, gsem0, gsem1, osem0, osem1):
    cid = lax.axis_index("c")
    sid = lax.axis_index("s")
    wid = sid * 2 + cid
    gsems = (gsem0, gsem1)
    osems = (osem0, osem1)

    pltpu.sync_copy(ids_hbm.at[:, pl.ds(wid * 512, 512)], ids_v)

    iota = lax.iota(jnp.int32, 16)
    cdt = tuple(lax.shift_right_logical(h * 16 + iota, 3) for h in range(2))
    clo = tuple(lax.bitwise_and(h * 16 + iota, 7) * 128 for h in range(2))

    def stage_a(f, p):
        @plsc.parallel_loop(0, 32, unroll=4)
        def _idx(i):
            sl = pl.ds(i * 16, 16)
            v = ids_v[f, sl]
            idx1_v[p, sl] = lax.bitwise_and(v, jnp.int32(65535)) + jnp.int32(_C0)
            h = lax.shift_right_logical(v, 16)
            x = h * jnp.int32(1024) + jnp.where(
                h >= jnp.int32(32768), jnp.int32(511), jnp.int32(0))
            q = (x.astype(jnp.float32) * jnp.float32(1.0 / 15625)).astype(jnp.int32)
            r = x - q * jnp.int32(15625)
            r = jnp.where(r < 0, r + jnp.int32(15625), r)
            r = jnp.where(r >= jnp.int32(15625), r - jnp.int32(15625), r)
            idx0_v[p, sl] = r

        for j in range(4):
            jsl = pl.ds(j * 128, 128)
            pltpu.async_copy(ctab_hbm.at[idx0_v.at[p, jsl]], buf0.at[p, jsl],
                             gsems[p])
            pltpu.async_copy(ctab_hbm.at[idx1_v.at[p, jsl]], buf1.at[p, jsl],
                             gsems[p])

    def stage_b(f, p, first):
        for j in range(4):
            jsl = pl.ds(j * 128, 128)
            pltpu.make_async_copy(ctab_hbm.at[pl.ds(0, 128)], buf0.at[p, jsl],
                                  gsems[p]).wait()
            pltpu.make_async_copy(ctab_hbm.at[pl.ds(0, 128)], buf1.at[p, jsl],
                                  gsems[p]).wait()

        def drain_out():
            for dt in range(4):
                pltpu.make_async_copy(out_hbm.at[0, 0, pl.ds(0, _BPW)],
                                      vout.at[p, dt], osems[p]).wait()

        if first is None:
            drain_out()
        else:
            pl.when(first)(drain_out)

        @plsc.parallel_loop(0, 512, unroll=8)
        def _asm(r_):
            btl = jnp.full((16,), 0, jnp.int32) + lax.shift_right_logical(r_, 7)
            bi = jnp.full((16,), 0, jnp.int32) + lax.bitwise_and(r_, 127)
            for half in range(2):
                sl = pl.ds(half * 16, 16)
                s = buf0[p, r_, sl] + buf1[p, r_, sl]
                plsc.store_scatter(vout.at[p], [cdt[half], btl, clo[half] + bi], s)

        for dt in range(4):
            pltpu.async_copy(vout.at[p, dt],
                             out_hbm.at[f, dt, pl.ds(wid * _BPW, _BPW)],
                             osems[p])

    stage_a(jnp.int32(0), 0)
    stage_a(jnp.int32(1), 1)

    def outer(go, carry):
        f0 = 2 * go
        for k in range(2):
            stage_b(f0 + k, k, go > 0)
            stage_a(f0 + k + 2, k)
        return carry

    lax.fori_loop(0, (_F - 2) // 2, outer, 0)

    stage_b(jnp.int32(24), 0, None)
    stage_b(jnp.int32(25), 1, None)
    for p in (0, 1):
        for dt in range(4):
            pltpu.make_async_copy(out_hbm.at[0, 0, pl.ds(0, _BPW)],
                                  vout.at[p, dt], osems[p]).wait()


def kernel(ids, table):
    c0 = table[::64]
    c1 = table[:_C1]
    pad = jnp.zeros((_CC - _C0 - _C1, _D), jnp.float32)
    cc = jnp.concatenate([c0, c1, pad], axis=0)
    lin = _relayout(cc.T)
    out4 = _gather(ids.T, lin.reshape(_CC, _D))
    out5 = out4.reshape(_F, 4, 128, 8, 128)
    return out5.transpose(2, 4, 0, 1, 3).reshape(_B, _F, _D)


# speedup vs baseline: 1.5496x; 1.5463x over previous
import functools

import jax
import jax.numpy as jnp
from jax import lax
from jax.experimental import pallas as pl
from jax.experimental.pallas import tpu as pltpu
from jax.experimental.pallas import tpu_sc as plsc

_D = 32
_B = 16384
_F = 26
_NW = 32
_C0 = 15625
_C1 = 65536
_CC = 81280
_VB = _CC // 128
_BT = _B // 128
_UNITS = _F * _BT
_UPW = _UNITS // _NW
_BPW = 4

_mesh = plsc.VectorSubcoreMesh(core_axis_name="c", subcore_axis_name="s")


@functools.partial(
    pl.kernel,
    out_type=jax.ShapeDtypeStruct((_CC * _D // 128, 128), jnp.float32),
    mesh=_mesh,
    compiler_params=pltpu.CompilerParams(use_tc_tiling_on_sc=True, needs_layout_passes=False, disable_bounds_checks=True),
    scratch_types=[
        pltpu.VMEM((2, 4, 8, 128), jnp.float32),
        pltpu.VMEM((2, 32, 128), jnp.float32),
        pltpu.SemaphoreType.DMA,
        pltpu.SemaphoreType.DMA,
        pltpu.SemaphoreType.DMA,
        pltpu.SemaphoreType.DMA,
    ],
)
def _relayout(src_hbm, lin_hbm, vin, vout, gsem0, gsem1, osem0, osem1):
    cid = lax.axis_index("c")
    sid = lax.axis_index("s")
    wid = sid * 2 + cid
    gsems = (gsem0, gsem1)
    osems = (osem0, osem1)
    nfull = 19

    def fire(n):
        p = n % 2
        vb = wid + 32 * n
        for dt in range(4):
            pltpu.async_copy(
                src_hbm.at[pl.ds(dt * 8, 8), pl.ds(vb * 128, 128)],
                vin.at[p, dt], gsems[p])

    def retire(n):
        p = n % 2
        vb = wid + 32 * n
        for dt in range(4):
            pltpu.make_async_copy(lin_hbm.at[pl.ds(0, 8)], vin.at[p, dt],
                                  gsems[p]).wait()
        if n >= 2:
            pltpu.make_async_copy(lin_hbm.at[pl.ds(0, 32)], vout.at[p],
                                  osems[p]).wait()

        @plsc.parallel_loop(0, 256, unroll=4)
        def _asm(j):
            mr = lax.shift_right_logical(j, 3)
            jj = lax.bitwise_and(j, 7)
            lane0 = jj * 16
            d = lax.bitwise_and(lane0, 31) + lax.iota(jnp.int32, 16)
            dt_v = lax.shift_right_logical(d, 3)
            di_v = lax.bitwise_and(d, 7)
            vi = jnp.full((16,), 0, jnp.int32) + (mr * 4 + lax.shift_right_logical(jj, 1))
            vout[p, mr, pl.ds(lane0, 16)] = plsc.load_gather(
                vin.at[p], [dt_v, di_v, vi])

        pltpu.async_copy(vout.at[p], lin_hbm.at[pl.ds(vb * 32, 32)], osems[p])

    fire(0)
    for n in range(1, nfull):
        fire(n)
        retire(n - 1)

    @pl.when(wid < 27)
    def _():
        fire(19)
    retire(nfull - 1)

    @pl.when(wid < 27)
    def _():
        retire(19)

    pltpu.make_async_copy(lin_hbm.at[pl.ds(0, 32)], vout.at[0],
                          osems[0]).wait()
    pltpu.make_async_copy(lin_hbm.at[pl.ds(0, 32)], vout.at[1],
                          osems[1]).wait()


@functools.partial(
    pl.kernel,
    out_type=jax.ShapeDtypeStruct((_F, 4, 128, 8, 128), jnp.float32),
    mesh=_mesh,
    compiler_params=pltpu.CompilerParams(use_tc_tiling_on_sc=False, needs_layout_passes=False, disable_bounds_checks=True),
    scratch_types=[
        pltpu.VMEM((_F, 512), jnp.int32),
        pltpu.VMEM((2, 512), jnp.int32),
        pltpu.VMEM((2, 512), jnp.int32),
        pltpu.VMEM((2, 512, _D), jnp.float32),
        pltpu.VMEM((2, 512, _D), jnp.float32),
        pltpu.VMEM((2, 4, 4, 8, 129), jnp.float32),
        pltpu.SemaphoreType.DMA,
        pltpu.SemaphoreType.DMA,
        pltpu.SemaphoreType.DMA,
        pltpu.SemaphoreType.DMA,
    ],
)
def _gather(ids_hbm, ctab_hbm, out_hbm, ids_v, idx0_v, idx1_v, buf0, buf1,
            vout, gsem0, gsem1, osem0, osem1):
    cid = lax.axis_index("c")
    sid = lax.axis_index("s")
    wid = sid * 2 + cid
    gsems = (gsem0, gsem1)
    osems = (osem0, osem1)

    pltpu.sync_copy(ids_hbm.at[:, pl.ds(wid * 512, 512)], ids_v)

    iota = lax.iota(jnp.int32, 16)
    cdt = tuple(lax.shift_right_logical(h * 16 + iota, 3) for h in range(2))
    cdi = tuple(lax.bitwise_and(h * 16 + iota, 7) for h in range(2))

    def stage_a(f, p):
        @plsc.parallel_loop(0, 32, unroll=4)
        def _idx(i):
            sl = pl.ds(i * 16, 16)
            v = ids_v[f, sl]
            idx1_v[p, sl] = lax.bitwise_and(v, jnp.int32(65535)) + jnp.int32(_C0)
            h = lax.shift_right_logical(v, 16)
            x = h * jnp.int32(1024) + jnp.where(
                h >= jnp.int32(32768), jnp.int32(511), jnp.int32(0))
            q = (x.astype(jnp.float32) * jnp.float32(1.0 / 15625)).astype(jnp.int32)
            r = x - q * jnp.int32(15625)
            r = jnp.where(r < 0, r + jnp.int32(15625), r)
            r = jnp.where(r >= jnp.int32(15625), r - jnp.int32(15625), r)
            idx0_v[p, sl] = r

        for j in range(4):
            jsl = pl.ds(j * 128, 128)
            pltpu.async_copy(ctab_hbm.at[idx0_v.at[p, jsl]], buf0.at[p, jsl],
                             gsems[p])
            pltpu.async_copy(ctab_hbm.at[idx1_v.at[p, jsl]], buf1.at[p, jsl],
                             gsems[p])

    def stage_b(f, p, first):
        for j in range(4):
            jsl = pl.ds(j * 128, 128)
            pltpu.make_async_copy(ctab_hbm.at[pl.ds(0, 128)], buf0.at[p, jsl],
                                  gsems[p]).wait()
            pltpu.make_async_copy(ctab_hbm.at[pl.ds(0, 128)], buf1.at[p, jsl],
                                  gsems[p]).wait()

        def drain_out():
            for dt in range(4):
                pltpu.make_async_copy(ctab_hbm.at[pl.ds(0, 128)],
                                      buf0.at[p, pl.ds(0, 128)],
                                      osems[p]).wait()

        if first is None:
            drain_out()
        else:
            pl.when(first)(drain_out)

        @plsc.parallel_loop(0, 512, unroll=8)
        def _asm(r_):
            btl = jnp.full((16,), 0, jnp.int32) + lax.shift_right_logical(r_, 7)
            bi = jnp.full((16,), 0, jnp.int32) + lax.bitwise_and(r_, 127)
            for half in range(2):
                sl = pl.ds(half * 16, 16)
                s = buf0[p, r_, sl] + buf1[p, r_, sl]
                plsc.store_scatter(vout.at[p], [cdt[half], btl, cdi[half], bi],
                                   s)

        for dt in range(4):
            for btl in range(4):
                pltpu.async_copy(
                    vout.at[p, dt, btl, pl.ds(0, 8), pl.ds(0, 128)],
                    out_hbm.at[f, dt, wid * _BPW + btl], osems[p])

    stage_a(jnp.int32(0), 0)
    stage_a(jnp.int32(1), 1)

    def outer(go, carry):
        f0 = 2 * go
        for k in range(2):
            stage_b(f0 + k, k, go > 0)
            stage_a(f0 + k + 2, k)
        return carry

    lax.fori_loop(0, (_F - 2) // 2, outer, 0)

    stage_b(jnp.int32(24), 0, None)
    stage_b(jnp.int32(25), 1, None)
    for p in (0, 1):
        for dt in range(4):
            pltpu.make_async_copy(ctab_hbm.at[pl.ds(0, 128)],
                                  buf0.at[p, pl.ds(0, 128)], osems[p]).wait()


def kernel(ids, table):
    c0 = table[::64]
    c1 = table[:_C1]
    pad = jnp.zeros((_CC - _C0 - _C1, _D), jnp.float32)
    cc = jnp.concatenate([c0, c1, pad], axis=0)
    lin = _relayout(cc.T)
    out5 = _gather(ids.T, lin.reshape(_CC, _D))
    return out5.transpose(2, 4, 0, 1, 3).reshape(_B, _F, _D)
